# baseline (speedup 1.0000x reference)
import functools

import jax
import jax.numpy as jnp
from jax import lax
from jax.experimental import pallas as pl
from jax.experimental.pallas import tpu as pltpu
from jax.experimental.pallas import tpu_sc as plsc

NC = 2
NS = 16
NW = NC * NS

H = 72
HP = 128


def _sc_mesh():
    return plsc.VectorSubcoreMesh(
        core_axis_name="c", subcore_axis_name="s", num_cores=NC, num_subcores=NS
    )


def _deg_call(dst2d, npad):
    nrows = dst2d.shape[0]
    rows_t = nrows // NW
    slice_n = npad // NS

    @functools.partial(
        pl.kernel,
        out_type=jax.ShapeDtypeStruct((NC, npad, HP), jnp.float32),
        mesh=_sc_mesh(),
        scratch_types=[
            pltpu.VMEM((128,), jnp.int32),
            pltpu.VMEM((128, HP), jnp.float32),
            pltpu.VMEM((128, HP), jnp.float32),
            pltpu.VMEM_SHARED((npad, HP), jnp.float32),
            pltpu.SemaphoreType.DMA,
        ],
    )
    def k(dst_hbm, out_hbm, idx_v, ones_v, zero_v, acc_sh, sem):
        c = lax.axis_index("c")
        s = lax.axis_index("s")
        wid = c * NS + s

        @pl.loop(0, 128)
        def _(i):
            for kk in range(HP // 16):
                ones_v[i, pl.ds(kk * 16, 16)] = jnp.ones((16,), jnp.float32)
                zero_v[i, pl.ds(kk * 16, 16)] = jnp.zeros((16,), jnp.float32)

        @pl.loop(0, slice_n // 128)
        def _(j):
            pltpu.sync_copy(zero_v, acc_sh.at[pl.ds(s * slice_n + j * 128, 128)])

        plsc.subcore_barrier()

        @pl.loop(0, rows_t)
        def _(g):
            pltpu.sync_copy(dst_hbm.at[wid * rows_t + g], idx_v)
            pltpu.sync_copy(ones_v, acc_sh.at[idx_v], add=True)

        plsc.subcore_barrier()

        @pl.loop(0, slice_n // 128)
        def _(j):
            base = s * slice_n + j * 128
            pltpu.sync_copy(acc_sh.at[pl.ds(base, 128)],
                            out_hbm.at[c, pl.ds(base, 128)])

    return k(dst2d)


def _segsum_call(src2d, dst2d, table, npad):
    nrows = src2d.shape[0]
    rows_t = nrows // NW
    iters = rows_t // 2
    slice_n = npad // NS

    @functools.partial(
        pl.kernel,
        out_type=jax.ShapeDtypeStruct((NC, npad, HP), jnp.float32),
        mesh=_sc_mesh(),
        scratch_types=[
            pltpu.VMEM((8, 128), jnp.int32),
            pltpu.VMEM((8, 128), jnp.int32),
            pltpu.VMEM((128, HP), jnp.float32),
            pltpu.VMEM((128, HP), jnp.float32),
            pltpu.VMEM((32, HP), jnp.float32),
            pltpu.VMEM_SHARED((npad, HP), jnp.float32),
            pltpu.SemaphoreType.DMA,
            pltpu.SemaphoreType.DMA,
        ],
    )
    def k(src_hbm, dst_hbm, tab_hbm, out_hbm, sidx, didx, rows_a, rows_b,
          zero_v, acc_sh, gsem, ssem):
        c = lax.axis_index("c")
        s = lax.axis_index("s")
        rows = [rows_a, rows_b]
        rt0 = ((rows_t * 2) // 4 // 8) * 8
        rt1 = rows_t * 2 - rt0
        rt_c = jnp.where(c == 0, rt0, rt1)
        tile_base = jnp.where(c == 0, s * rt0, NS * rt0 + s * rt1)

        @pl.loop(0, 32)
        def _(i):
            for kk in range(HP // 16):
                zero_v[i, pl.ds(kk * 16, 16)] = jnp.zeros((16,), jnp.float32)

        @pl.loop(0, slice_n // 32)
        def _(j):
            pltpu.sync_copy(zero_v, acc_sh.at[pl.ds(s * slice_n + j * 32, 32)])

        plsc.subcore_barrier()

        @pl.loop(0, rt_c // 8)
        def _(blk):
            base = tile_base + blk * 8
            pltpu.sync_copy(src_hbm.at[pl.ds(base, 8)], sidx)
            pltpu.sync_copy(dst_hbm.at[pl.ds(base, 8)], didx)
            gd = [None, None]
            sd = [None, None]
            gd[0] = pltpu.async_copy(tab_hbm.at[sidx.at[0]], rows[0], gsem)
            for j in range(8):
                b = j % 2
                if j >= 1:
                    sd[1 - b].wait()
                if j + 1 < 8:
                    gd[1 - b] = pltpu.async_copy(
                        tab_hbm.at[sidx.at[j + 1]], rows[1 - b], gsem)
                gd[b].wait()
                sd[b] = pltpu.async_copy(
                    rows[b], acc_sh.at[didx.at[j]], ssem, add=True)
            sd[1].wait()

        plsc.subcore_barrier()

        @pl.loop(0, slice_n // 128)
        def _(j):
            base = s * slice_n + j * 128
            pltpu.sync_copy(acc_sh.at[pl.ds(base, 128)],
                            out_hbm.at[c, pl.ds(base, 128)])

    return k(src2d, dst2d, table)


def _headgather_call(src2d, dst2d, ta, tb):
    nrows = src2d.shape[0]
    rows_t = nrows // NW

    @functools.partial(
        pl.kernel,
        out_type=(
            jax.ShapeDtypeStruct((nrows, 128, HP), jnp.float32),
            jax.ShapeDtypeStruct((nrows, 128, HP), jnp.float32),
        ),
        mesh=_sc_mesh(),
        scratch_types=[
            pltpu.VMEM((8, 128), jnp.int32),
            pltpu.VMEM((8, 128), jnp.int32),
            pltpu.VMEM((128, HP), jnp.float32),
            pltpu.VMEM((128, HP), jnp.float32),
            pltpu.VMEM((128, HP), jnp.float32),
            pltpu.VMEM((128, HP), jnp.float32),
            pltpu.SemaphoreType.DMA,
            pltpu.SemaphoreType.DMA,
        ],
    )
    def k(src_hbm, dst_hbm, ta_hbm, tb_hbm, g1_hbm, g2_hbm, sidx, didx,
          r1a, r1b, r2a, r2b, gsem, wsem):
        c = lax.axis_index("c")
        s = lax.axis_index("s")
        r1 = [r1a, r1b]
        r2 = [r2a, r2b]
        rt0 = ((rows_t * 2) // 4 // 8) * 8
        rt1 = rows_t * 2 - rt0
        rt_c = jnp.where(c == 0, rt0, rt1)
        tile_base = jnp.where(c == 0, s * rt0, NS * rt0 + s * rt1)

        @pl.loop(0, rt_c // 8)
        def _(blk):
            base = tile_base + blk * 8
            pltpu.sync_copy(src_hbm.at[pl.ds(base, 8)], sidx)
            pltpu.sync_copy(dst_hbm.at[pl.ds(base, 8)], didx)
            gd1 = [None, None]
            gd2 = [None, None]
            wd1 = [None, None]
            wd2 = [None, None]
            gd1[0] = pltpu.async_copy(ta_hbm.at[sidx.at[0]], r1[0], gsem)
            gd2[0] = pltpu.async_copy(tb_hbm.at[didx.at[0]], r2[0], gsem)
            for j in range(8):
                b = j % 2
                if j >= 1:
                    wd1[1 - b].wait()
                    wd2[1 - b].wait()
                if j + 1 < 8:
                    gd1[1 - b] = pltpu.async_copy(
                        ta_hbm.at[sidx.at[j + 1]], r1[1 - b], gsem)
                    gd2[1 - b] = pltpu.async_copy(
                        tb_hbm.at[didx.at[j + 1]], r2[1 - b], gsem)
                gd1[b].wait()
                gd2[b].wait()
                wd1[b] = pltpu.async_copy(r1[b], g1_hbm.at[base + j], wsem)
                wd2[b] = pltpu.async_copy(r2[b], g2_hbm.at[base + j], wsem)
            wd1[1].wait()
            wd2[1].wait()

    return k(src2d, dst2d, ta, tb)


def _tc_prep(degcol, x, Wn, bn, Wi0):
    n = x.shape[0]

    def body(deg_ref, x_ref, wn_ref, bn_ref, wi_ref, h_ref, dinv_ref, outs_ref):
        deg = deg_ref[...]
        dinv = jnp.where(deg > 0, lax.rsqrt(deg), 0.0)
        h = jnp.maximum(
            jnp.dot(x_ref[...], wn_ref[...], preferred_element_type=jnp.float32)
            + bn_ref[...],
            0.0,
        )
        h_ref[...] = h
        dinv_ref[...] = dinv
        outs_ref[:, :H] = dinv * jnp.dot(
            h, wi_ref[...], preferred_element_type=jnp.float32
        )
        outs_ref[:, H:] = jnp.zeros((n, HP - H), jnp.float32)

    return pl.pallas_call(
        body,
        out_shape=(
            jax.ShapeDtypeStruct((n, H), jnp.float32),
            jax.ShapeDtypeStruct((n, 1), jnp.float32),
            jax.ShapeDtypeStruct((n, HP), jnp.float32),
        ),
    )(degcol, x, Wn, bn, Wi0)


def _tc_layer(parts, h, dinv, Wr, bc, g, b, Wi_next):
    n = h.shape[0]

    def body(p_ref, h_ref, dinv_ref, wr_ref, bc_ref, g_ref, b_ref, wi_ref,
             hn_ref, outs_ref):
        dinv = dinv_ref[...]
        agg = dinv * (p_ref[0, :n, :H] + p_ref[1, :n, :H])
        pre = agg + jnp.dot(h_ref[...], wr_ref[...],
                            preferred_element_type=jnp.float32) + bc_ref[...]
        out = jnp.maximum(pre, 0.0)
        mean = jnp.mean(out, axis=0, keepdims=True)
        var = jnp.mean((out - mean) * (out - mean), axis=0, keepdims=True)
        hn = (out - mean) * lax.rsqrt(var + 1e-5) * g_ref[...] + b_ref[...]
        hn_ref[...] = hn
        outs_ref[:, :H] = dinv * jnp.dot(hn, wi_ref[...],
                                         preferred_element_type=jnp.float32)
        outs_ref[:, H:] = jnp.zeros((n, HP - H), jnp.float32)

    return pl.pallas_call(
        body,
        out_shape=(
            jax.ShapeDtypeStruct((n, H), jnp.float32),
            jax.ShapeDtypeStruct((n, HP), jnp.float32),
        ),
    )(parts, h, dinv, Wr, bc, g, b, Wi_next)


def _tc_layer_final(parts, h, dinv, Wr, bc, g, b, We1, Wm1):
    n = h.shape[0]

    def body(p_ref, h_ref, dinv_ref, wr_ref, bc_ref, g_ref, b_ref,
             we1_ref, wm1_ref, a_ref, b2_ref):
        dinv = dinv_ref[...]
        agg = dinv * (p_ref[0, :n, :H] + p_ref[1, :n, :H])
        pre = agg + jnp.dot(h_ref[...], wr_ref[...],
                            preferred_element_type=jnp.float32) + bc_ref[...]
        out = jnp.maximum(pre, 0.0)
        mean = jnp.mean(out, axis=0, keepdims=True)
        var = jnp.mean((out - mean) * (out - mean), axis=0, keepdims=True)
        hn = (out - mean) * lax.rsqrt(var + 1e-5) * g_ref[...] + b_ref[...]
        wa = jnp.dot(we1_ref[:H, :], wm1_ref[:H, :],
                     preferred_element_type=jnp.float32)
        wb = jnp.dot(we1_ref[H:, :], wm1_ref[:H, :],
                     preferred_element_type=jnp.float32)
        a_ref[:, :H] = jnp.dot(hn, wa, preferred_element_type=jnp.float32)
        a_ref[:, H:] = jnp.zeros((n, HP - H), jnp.float32)
        b2_ref[:, :H] = jnp.dot(hn, wb, preferred_element_type=jnp.float32)
        b2_ref[:, H:] = jnp.zeros((n, HP - H), jnp.float32)

    return pl.pallas_call(
        body,
        out_shape=(
            jax.ShapeDtypeStruct((n, HP), jnp.float32),
            jax.ShapeDtypeStruct((n, HP), jnp.float32),
        ),
    )(parts, h, dinv, Wr, bc, g, b, We1, Wm1)


def _tc_head(g1, g2, ea, We2, Wm1, be1, be2, bm1, Wm2, bm2):
    epad = g1.shape[0]
    BE = 4096
    d_edge = ea.shape[1]

    def body(g1_ref, g2_ref, ea_ref, we2_ref, wm1_ref, be1_ref, be2_ref,
             bm1_ref, wm2_ref, bm2_ref, out_ref):
        wc = jnp.dot(we2_ref[...], wm1_ref[H:, :],
                     preferred_element_type=jnp.float32)
        cb = (jnp.dot(be1_ref[...][None, :], wm1_ref[:H, :],
                      preferred_element_type=jnp.float32)
              + jnp.dot(be2_ref[...][None, :], wm1_ref[H:, :],
                        preferred_element_type=jnp.float32)
              + bm1_ref[...][None, :])
        z = (g1_ref[:, :H] + g2_ref[:, :H]
             + jnp.dot(ea_ref[...], wc, preferred_element_type=jnp.float32)
             + cb)
        out_ref[...] = (
            jnp.dot(jnp.maximum(z, 0.0), wm2_ref[...],
                    preferred_element_type=jnp.float32)
            + bm2_ref[...][None, :]
        )

    grid = (epad // BE,)
    return pl.pallas_call(
        body,
        grid=grid,
        in_specs=[
            pl.BlockSpec((BE, HP), lambda i: (i, 0)),
            pl.BlockSpec((BE, HP), lambda i: (i, 0)),
            pl.BlockSpec((BE, d_edge), lambda i: (i, 0)),
            pl.BlockSpec(We2.shape, lambda i: (0, 0)),
            pl.BlockSpec(Wm1.shape, lambda i: (0, 0)),
            pl.BlockSpec(be1.shape, lambda i: (0,)),
            pl.BlockSpec(be2.shape, lambda i: (0,)),
            pl.BlockSpec(bm1.shape, lambda i: (0,)),
            pl.BlockSpec(Wm2.shape, lambda i: (0, 0)),
            pl.BlockSpec(bm2.shape, lambda i: (0,)),
        ],
        out_specs=pl.BlockSpec((BE, 1), lambda i: (i, 0)),
        out_shape=jax.ShapeDtypeStruct((epad, 1), jnp.float32),
    )(g1, g2, ea, We2, Wm1, be1, be2, bm1, Wm2, bm2)


def kernel(x, EdgeID, EdgeAttr, Wn, bn, W_init, W_root, b_conv, gamma, beta,
           We1, be1, We2, be2, Wm1, bm1, Wm2, bm2):
    n = x.shape[0]
    e = EdgeID.shape[1]
    num_layers = W_init.shape[0]

    npad = ((n + NS * 128 - 1) // (NS * 128)) * (NS * 128)
    epad = ((e + NW * 1024 - 1) // (NW * 1024)) * (NW * 1024)

    src = EdgeID[0]
    dst = EdgeID[1]
    src_p = jnp.concatenate([src, jnp.zeros((epad - e,), jnp.int32)])
    dst_p = jnp.concatenate([dst, jnp.full((epad - e,), n, jnp.int32)])
    dst_g = jnp.concatenate([dst, jnp.zeros((epad - e,), jnp.int32)])
    src2d = src_p.reshape(-1, 128)
    dst2d = dst_p.reshape(-1, 128)
    dstg2d = dst_g.reshape(-1, 128)

    def _deg_jnp(dst2d_, npad_):
        d = dst2d_.reshape(-1)
        half = d.shape[0] // 2
        out = jnp.zeros((NC, npad_), jnp.float32)
        out = out.at[0, :].add(
            jnp.zeros((npad_,), jnp.float32).at[d[:half]].add(1.0))
        out = out.at[1, :].add(
            jnp.zeros((npad_,), jnp.float32).at[d[half:]].add(1.0))
        return out.reshape(-1)

    def _segsum_jnp(src2d_, dst2d_, table_, npad_):
        s_ = src2d_.reshape(-1)
        d_ = dst2d_.reshape(-1)
        half = s_.shape[0] // 2
        p0 = jnp.zeros((npad_, HP), jnp.float32).at[d_[:half]].add(table_[s_[:half]])
        p1 = jnp.zeros((npad_, HP), jnp.float32).at[d_[half:]].add(table_[s_[half:]])
        return jnp.stack([p0, p1])

    degp = _deg_call(dst2d, npad)
    degcol = (degp[0, :n, 0] + degp[1, :n, 0])[:, None]
    h, dinv, outs = _tc_prep(degcol, x, Wn, bn, W_init[0])

    for l in range(num_layers):
        parts = _segsum_call(src2d, dst2d, outs, npad)
        if l < num_layers - 1:
            h, outs = _tc_layer(parts, h, dinv, W_root[l], b_conv[l],
                                gamma[l], beta[l], W_init[l + 1])
        else:
            ta, tb = _tc_layer_final(parts, h, dinv, W_root[l], b_conv[l],
                                     gamma[l], beta[l], We1, Wm1)

    g1, g2 = _headgather_call(src2d, dstg2d, ta, tb)
    ea_p = jnp.pad(EdgeAttr, ((0, epad - e), (0, 0)))
    out = _tc_head(g1.reshape(epad, HP), g2.reshape(epad, HP), ea_p,
                   We2, Wm1, be1, be2, bm1, Wm2, bm2)
    return out[:e]


# speedup vs baseline: 1.0462x; 1.0462x over previous
import functools

import jax
import jax.numpy as jnp
from jax import lax
from jax.experimental import pallas as pl
from jax.experimental.pallas import tpu as pltpu
from jax.experimental.pallas import tpu_sc as plsc

NC = 2
NS = 16
NW = NC * NS

H = 72
HP = 128


def _sc_mesh():
    return plsc.VectorSubcoreMesh(
        core_axis_name="c", subcore_axis_name="s", num_cores=NC, num_subcores=NS
    )


def _deg_call(dst2d, npad):
    nrows = dst2d.shape[0]
    rows_t = nrows // NW
    slice_n = npad // NS

    @functools.partial(
        pl.kernel,
        out_type=jax.ShapeDtypeStruct((NC, npad, HP), jnp.float32),
        mesh=_sc_mesh(),
        scratch_types=[
            pltpu.VMEM((128,), jnp.int32),
            pltpu.VMEM((128, HP), jnp.float32),
            pltpu.VMEM((128, HP), jnp.float32),
            pltpu.VMEM_SHARED((npad, HP), jnp.float32),
            pltpu.SemaphoreType.DMA,
        ],
    )
    def k(dst_hbm, out_hbm, idx_v, ones_v, zero_v, acc_sh, sem):
        c = lax.axis_index("c")
        s = lax.axis_index("s")
        wid = c * NS + s

        @pl.loop(0, 128)
        def _(i):
            for kk in range(HP // 16):
                ones_v[i, pl.ds(kk * 16, 16)] = jnp.ones((16,), jnp.float32)
                zero_v[i, pl.ds(kk * 16, 16)] = jnp.zeros((16,), jnp.float32)

        @pl.loop(0, slice_n // 128)
        def _(j):
            pltpu.sync_copy(zero_v, acc_sh.at[pl.ds(s * slice_n + j * 128, 128)])

        plsc.subcore_barrier()

        @pl.loop(0, rows_t)
        def _(g):
            pltpu.sync_copy(dst_hbm.at[wid * rows_t + g], idx_v)
            pltpu.sync_copy(ones_v, acc_sh.at[idx_v], add=True)

        plsc.subcore_barrier()

        @pl.loop(0, slice_n // 128)
        def _(j):
            base = s * slice_n + j * 128
            pltpu.sync_copy(acc_sh.at[pl.ds(base, 128)],
                            out_hbm.at[c, pl.ds(base, 128)])

    return k(dst2d)


def _segsum_call(src2d, dst2d, table, npad):
    nrows = src2d.shape[0]
    rows_t = nrows // NW
    iters = rows_t // 2
    slice_n = npad // NS

    @functools.partial(
        pl.kernel,
        out_type=jax.ShapeDtypeStruct((NC, npad, HP), jnp.float32),
        mesh=_sc_mesh(),
        scratch_types=[
            pltpu.VMEM((8, 128), jnp.int32),
            pltpu.VMEM((8, 128), jnp.int32),
            pltpu.VMEM((128, HP), jnp.float32),
            pltpu.VMEM((128, HP), jnp.float32),
            pltpu.VMEM((32, HP), jnp.float32),
            pltpu.VMEM_SHARED((npad, HP), jnp.float32),
            pltpu.SemaphoreType.DMA,
            pltpu.SemaphoreType.DMA,
        ],
    )
    def k(src_hbm, dst_hbm, tab_hbm, out_hbm, sidx, didx, rows_a, rows_b,
          zero_v, acc_sh, gsem, ssem):
        c = lax.axis_index("c")
        s = lax.axis_index("s")
        rows = [rows_a, rows_b]
        rt0 = ((rows_t * 2) // 4 // 8) * 8
        rt1 = rows_t * 2 - rt0
        rt_c = jnp.where(c == 1, rt0, rt1)
        tile_base = jnp.where(c == 1, s * rt0, NS * rt0 + s * rt1)

        @pl.loop(0, 32)
        def _(i):
            for kk in range(HP // 16):
                zero_v[i, pl.ds(kk * 16, 16)] = jnp.zeros((16,), jnp.float32)

        @pl.loop(0, slice_n // 32)
        def _(j):
            pltpu.sync_copy(zero_v, acc_sh.at[pl.ds(s * slice_n + j * 32, 32)])

        plsc.subcore_barrier()

        @pl.loop(0, rt_c // 8)
        def _(blk):
            base = tile_base + blk * 8
            pltpu.sync_copy(src_hbm.at[pl.ds(base, 8)], sidx)
            pltpu.sync_copy(dst_hbm.at[pl.ds(base, 8)], didx)
            gd = [None, None]
            sd = [None, None]
            gd[0] = pltpu.async_copy(tab_hbm.at[sidx.at[0]], rows[0], gsem)
            for j in range(8):
                b = j % 2
                if j >= 1:
                    sd[1 - b].wait()
                if j + 1 < 8:
                    gd[1 - b] = pltpu.async_copy(
                        tab_hbm.at[sidx.at[j + 1]], rows[1 - b], gsem)
                gd[b].wait()
                sd[b] = pltpu.async_copy(
                    rows[b], acc_sh.at[didx.at[j]], ssem, add=True)
            sd[1].wait()

        plsc.subcore_barrier()

        @pl.loop(0, slice_n // 128)
        def _(j):
            base = s * slice_n + j * 128
            pltpu.sync_copy(acc_sh.at[pl.ds(base, 128)],
                            out_hbm.at[c, pl.ds(base, 128)])

    return k(src2d, dst2d, table)


def _headgather_call(src2d, dst2d, ta, tb):
    nrows = src2d.shape[0]
    rows_t = nrows // NW

    @functools.partial(
        pl.kernel,
        out_type=(
            jax.ShapeDtypeStruct((nrows, 128, HP), jnp.float32),
            jax.ShapeDtypeStruct((nrows, 128, HP), jnp.float32),
        ),
        mesh=_sc_mesh(),
        scratch_types=[
            pltpu.VMEM((8, 128), jnp.int32),
            pltpu.VMEM((8, 128), jnp.int32),
            pltpu.VMEM((128, HP), jnp.float32),
            pltpu.VMEM((128, HP), jnp.float32),
            pltpu.VMEM((128, HP), jnp.float32),
            pltpu.VMEM((128, HP), jnp.float32),
            pltpu.SemaphoreType.DMA,
            pltpu.SemaphoreType.DMA,
        ],
    )
    def k(src_hbm, dst_hbm, ta_hbm, tb_hbm, g1_hbm, g2_hbm, sidx, didx,
          r1a, r1b, r2a, r2b, gsem, wsem):
        c = lax.axis_index("c")
        s = lax.axis_index("s")
        r1 = [r1a, r1b]
        r2 = [r2a, r2b]
        rt0 = ((rows_t * 2) // 4 // 8) * 8
        rt1 = rows_t * 2 - rt0
        rt_c = jnp.where(c == 1, rt0, rt1)
        tile_base = jnp.where(c == 1, s * rt0, NS * rt0 + s * rt1)

        @pl.loop(0, rt_c // 8)
        def _(blk):
            base = tile_base + blk * 8
            pltpu.sync_copy(src_hbm.at[pl.ds(base, 8)], sidx)
            pltpu.sync_copy(dst_hbm.at[pl.ds(base, 8)], didx)
            gd1 = [None, None]
            gd2 = [None, None]
            wd1 = [None, None]
            wd2 = [None, None]
            gd1[0] = pltpu.async_copy(ta_hbm.at[sidx.at[0]], r1[0], gsem)
            gd2[0] = pltpu.async_copy(tb_hbm.at[didx.at[0]], r2[0], gsem)
            for j in range(8):
                b = j % 2
                if j >= 1:
                    wd1[1 - b].wait()
                    wd2[1 - b].wait()
                if j + 1 < 8:
                    gd1[1 - b] = pltpu.async_copy(
                        ta_hbm.at[sidx.at[j + 1]], r1[1 - b], gsem)
                    gd2[1 - b] = pltpu.async_copy(
                        tb_hbm.at[didx.at[j + 1]], r2[1 - b], gsem)
                gd1[b].wait()
                gd2[b].wait()
                wd1[b] = pltpu.async_copy(r1[b], g1_hbm.at[base + j], wsem)
                wd2[b] = pltpu.async_copy(r2[b], g2_hbm.at[base + j], wsem)
            wd1[1].wait()
            wd2[1].wait()

    return k(src2d, dst2d, ta, tb)


def _tc_prep(degcol, x, Wn, bn, Wi0):
    n = x.shape[0]

    def body(deg_ref, x_ref, wn_ref, bn_ref, wi_ref, h_ref, dinv_ref, outs_ref):
        deg = deg_ref[...]
        dinv = jnp.where(deg > 0, lax.rsqrt(deg), 0.0)
        h = jnp.maximum(
            jnp.dot(x_ref[...], wn_ref[...], preferred_element_type=jnp.float32)
            + bn_ref[...],
            0.0,
        )
        h_ref[...] = h
        dinv_ref[...] = dinv
        outs_ref[:, :H] = dinv * jnp.dot(
            h, wi_ref[...], preferred_element_type=jnp.float32
        )
        outs_ref[:, H:] = jnp.zeros((n, HP - H), jnp.float32)

    return pl.pallas_call(
        body,
        out_shape=(
            jax.ShapeDtypeStruct((n, H), jnp.float32),
            jax.ShapeDtypeStruct((n, 1), jnp.float32),
            jax.ShapeDtypeStruct((n, HP), jnp.float32),
        ),
    )(degcol, x, Wn, bn, Wi0)


def _tc_layer(parts, h, dinv, Wr, bc, g, b, Wi_next):
    n = h.shape[0]

    def body(p_ref, h_ref, dinv_ref, wr_ref, bc_ref, g_ref, b_ref, wi_ref,
             hn_ref, outs_ref):
        dinv = dinv_ref[...]
        agg = dinv * (p_ref[0, :n, :H] + p_ref[1, :n, :H])
        pre = agg + jnp.dot(h_ref[...], wr_ref[...],
                            preferred_element_type=jnp.float32) + bc_ref[...]
        out = jnp.maximum(pre, 0.0)
        mean = jnp.mean(out, axis=0, keepdims=True)
        var = jnp.mean((out - mean) * (out - mean), axis=0, keepdims=True)
        hn = (out - mean) * lax.rsqrt(var + 1e-5) * g_ref[...] + b_ref[...]
        hn_ref[...] = hn
        outs_ref[:, :H] = dinv * jnp.dot(hn, wi_ref[...],
                                         preferred_element_type=jnp.float32)
        outs_ref[:, H:] = jnp.zeros((n, HP - H), jnp.float32)

    return pl.pallas_call(
        body,
        out_shape=(
            jax.ShapeDtypeStruct((n, H), jnp.float32),
            jax.ShapeDtypeStruct((n, HP), jnp.float32),
        ),
    )(parts, h, dinv, Wr, bc, g, b, Wi_next)


def _tc_layer_final(parts, h, dinv, Wr, bc, g, b, We1, Wm1):
    n = h.shape[0]

    def body(p_ref, h_ref, dinv_ref, wr_ref, bc_ref, g_ref, b_ref,
             we1_ref, wm1_ref, a_ref, b2_ref):
        dinv = dinv_ref[...]
        agg = dinv * (p_ref[0, :n, :H] + p_ref[1, :n, :H])
        pre = agg + jnp.dot(h_ref[...], wr_ref[...],
                            preferred_element_type=jnp.float32) + bc_ref[...]
        out = jnp.maximum(pre, 0.0)
        mean = jnp.mean(out, axis=0, keepdims=True)
        var = jnp.mean((out - mean) * (out - mean), axis=0, keepdims=True)
        hn = (out - mean) * lax.rsqrt(var + 1e-5) * g_ref[...] + b_ref[...]
        wa = jnp.dot(we1_ref[:H, :], wm1_ref[:H, :],
                     preferred_element_type=jnp.float32)
        wb = jnp.dot(we1_ref[H:, :], wm1_ref[:H, :],
                     preferred_element_type=jnp.float32)
        a_ref[:, :H] = jnp.dot(hn, wa, preferred_element_type=jnp.float32)
        a_ref[:, H:] = jnp.zeros((n, HP - H), jnp.float32)
        b2_ref[:, :H] = jnp.dot(hn, wb, preferred_element_type=jnp.float32)
        b2_ref[:, H:] = jnp.zeros((n, HP - H), jnp.float32)

    return pl.pallas_call(
        body,
        out_shape=(
            jax.ShapeDtypeStruct((n, HP), jnp.float32),
            jax.ShapeDtypeStruct((n, HP), jnp.float32),
        ),
    )(parts, h, dinv, Wr, bc, g, b, We1, Wm1)


def _tc_head(g1, g2, ea, We2, Wm1, be1, be2, bm1, Wm2, bm2):
    epad = g1.shape[0]
    BE = 4096
    d_edge = ea.shape[1]

    def body(g1_ref, g2_ref, ea_ref, we2_ref, wm1_ref, be1_ref, be2_ref,
             bm1_ref, wm2_ref, bm2_ref, out_ref):
        wc = jnp.dot(we2_ref[...], wm1_ref[H:, :],
                     preferred_element_type=jnp.float32)
        cb = (jnp.dot(be1_ref[...][None, :], wm1_ref[:H, :],
                      preferred_element_type=jnp.float32)
              + jnp.dot(be2_ref[...][None, :], wm1_ref[H:, :],
                        preferred_element_type=jnp.float32)
              + bm1_ref[...][None, :])
        z = (g1_ref[:, :H] + g2_ref[:, :H]
             + jnp.dot(ea_ref[...], wc, preferred_element_type=jnp.float32)
             + cb)
        out_ref[...] = (
            jnp.dot(jnp.maximum(z, 0.0), wm2_ref[...],
                    preferred_element_type=jnp.float32)
            + bm2_ref[...][None, :]
        )

    grid = (epad // BE,)
    return pl.pallas_call(
        body,
        grid=grid,
        in_specs=[
            pl.BlockSpec((BE, HP), lambda i: (i, 0)),
            pl.BlockSpec((BE, HP), lambda i: (i, 0)),
            pl.BlockSpec((BE, d_edge), lambda i: (i, 0)),
            pl.BlockSpec(We2.shape, lambda i: (0, 0)),
            pl.BlockSpec(Wm1.shape, lambda i: (0, 0)),
            pl.BlockSpec(be1.shape, lambda i: (0,)),
            pl.BlockSpec(be2.shape, lambda i: (0,)),
            pl.BlockSpec(bm1.shape, lambda i: (0,)),
            pl.BlockSpec(Wm2.shape, lambda i: (0, 0)),
            pl.BlockSpec(bm2.shape, lambda i: (0,)),
        ],
        out_specs=pl.BlockSpec((BE, 1), lambda i: (i, 0)),
        out_shape=jax.ShapeDtypeStruct((epad, 1), jnp.float32),
    )(g1, g2, ea, We2, Wm1, be1, be2, bm1, Wm2, bm2)


def kernel(x, EdgeID, EdgeAttr, Wn, bn, W_init, W_root, b_conv, gamma, beta,
           We1, be1, We2, be2, Wm1, bm1, Wm2, bm2):
    n = x.shape[0]
    e = EdgeID.shape[1]
    num_layers = W_init.shape[0]

    npad = ((n + NS * 128 - 1) // (NS * 128)) * (NS * 128)
    epad = ((e + NW * 1024 - 1) // (NW * 1024)) * (NW * 1024)

    src = EdgeID[0]
    dst = EdgeID[1]
    src_p = jnp.concatenate([src, jnp.zeros((epad - e,), jnp.int32)])
    dst_p = jnp.concatenate([dst, jnp.full((epad - e,), n, jnp.int32)])
    dst_g = jnp.concatenate([dst, jnp.zeros((epad - e,), jnp.int32)])
    src2d = src_p.reshape(-1, 128)
    dst2d = dst_p.reshape(-1, 128)
    dstg2d = dst_g.reshape(-1, 128)

    def _deg_jnp(dst2d_, npad_):
        d = dst2d_.reshape(-1)
        half = d.shape[0] // 2
        out = jnp.zeros((NC, npad_), jnp.float32)
        out = out.at[0, :].add(
            jnp.zeros((npad_,), jnp.float32).at[d[:half]].add(1.0))
        out = out.at[1, :].add(
            jnp.zeros((npad_,), jnp.float32).at[d[half:]].add(1.0))
        return out.reshape(-1)

    def _segsum_jnp(src2d_, dst2d_, table_, npad_):
        s_ = src2d_.reshape(-1)
        d_ = dst2d_.reshape(-1)
        half = s_.shape[0] // 2
        p0 = jnp.zeros((npad_, HP), jnp.float32).at[d_[:half]].add(table_[s_[:half]])
        p1 = jnp.zeros((npad_, HP), jnp.float32).at[d_[half:]].add(table_[s_[half:]])
        return jnp.stack([p0, p1])

    degp = _deg_call(dst2d, npad)
    degcol = (degp[0, :n, 0] + degp[1, :n, 0])[:, None]
    h, dinv, outs = _tc_prep(degcol, x, Wn, bn, W_init[0])

    for l in range(num_layers):
        parts = _segsum_call(src2d, dst2d, outs, npad)
        if l < num_layers - 1:
            h, outs = _tc_layer(parts, h, dinv, W_root[l], b_conv[l],
                                gamma[l], beta[l], W_init[l + 1])
        else:
            ta, tb = _tc_layer_final(parts, h, dinv, W_root[l], b_conv[l],
                                     gamma[l], beta[l], We1, Wm1)

    g1, g2 = _headgather_call(src2d, dstg2d, ta, tb)
    ea_p = jnp.pad(EdgeAttr, ((0, epad - e), (0, 0)))
    out = _tc_head(g1.reshape(epad, HP), g2.reshape(epad, HP), ea_p,
                   We2, Wm1, be1, be2, bm1, Wm2, bm2)
    return out[:e]
